# baseline (speedup 1.0000x reference)
import functools

import jax
import jax.numpy as jnp
from jax.experimental import pallas as pl

BETA = 0.7


def _block_kernel(x1_ref, x2_ref, v_ref, out_ref):
    x1 = x1_ref[...]
    x2 = x2_ref[...]
    v = v_ref[...]

    m1 = jnp.max(x1, axis=1, keepdims=True)
    m2 = jnp.max(x2, axis=1, keepdims=True)
    e1 = jnp.exp(x1 - m1)
    e2 = jnp.exp(x2 - m2)
    s1 = jnp.sum(e1, axis=1, keepdims=True)
    s2 = jnp.sum(e2, axis=1, keepdims=True)

    beta = jnp.float32(BETA)
    p = (beta / s1) * e1 + ((1.0 - beta) / s2) * e2
    ent = -jnp.sum(p * jnp.log(p), axis=1)

    cols = jax.lax.broadcasted_iota(jnp.int32, x1.shape, 1)
    mask = cols == v
    g1 = jnp.sum(jnp.where(mask, x1, 0.0), axis=1)
    g2 = jnp.sum(jnp.where(mask, x2, 0.0), axis=1)

    lp1 = g1 - m1[:, 0] - jnp.log(s1[:, 0]) + jnp.log(beta)
    lp2 = g2 - m2[:, 0] - jnp.log(s2[:, 0]) + jnp.log(1.0 - beta)
    log_prob = jnp.logaddexp(lp1, lp2)

    out_ref[...] = jnp.concatenate([ent[:, None], log_prob[:, None]], axis=1)


@jax.jit
def kernel(logits_1, logits_2, value):
    B, V = logits_1.shape
    R = 8
    grid = (B // R,)
    v2d = value.astype(jnp.int32).reshape(B, 1)
    out = pl.pallas_call(
        _block_kernel,
        grid=grid,
        in_specs=[
            pl.BlockSpec((R, V), lambda i: (i, 0)),
            pl.BlockSpec((R, V), lambda i: (i, 0)),
            pl.BlockSpec((R, 1), lambda i: (i, 0)),
        ],
        out_specs=pl.BlockSpec((R, 2), lambda i: (i, 0)),
        out_shape=jax.ShapeDtypeStruct((B, 2), jnp.float32),
    )(logits_1, logits_2, v2d)
    return out


# speedup vs baseline: 1.0570x; 1.0570x over previous
import functools

import jax
import jax.numpy as jnp
from jax.experimental import pallas as pl

BETA = 0.7


def _block_kernel(x1_ref, x2_ref, v_ref, out_ref):
    x1 = x1_ref[...]
    x2 = x2_ref[...]
    v = v_ref[...]

    m1 = jnp.max(x1, axis=1, keepdims=True)
    m2 = jnp.max(x2, axis=1, keepdims=True)
    e1 = jnp.exp(x1 - m1)
    e2 = jnp.exp(x2 - m2)
    s1 = jnp.sum(e1, axis=1, keepdims=True)
    s2 = jnp.sum(e2, axis=1, keepdims=True)

    beta = jnp.float32(BETA)
    p = (beta / s1) * e1 + ((1.0 - beta) / s2) * e2
    ent = -jnp.sum(p * jnp.log(p), axis=1)

    cols = jax.lax.broadcasted_iota(jnp.int32, x1.shape, 1)
    mask = cols == v
    g1 = jnp.sum(jnp.where(mask, x1, 0.0), axis=1)
    g2 = jnp.sum(jnp.where(mask, x2, 0.0), axis=1)

    lp1 = g1 - m1[:, 0] - jnp.log(s1[:, 0]) + jnp.log(beta)
    lp2 = g2 - m2[:, 0] - jnp.log(s2[:, 0]) + jnp.log(1.0 - beta)
    log_prob = jnp.logaddexp(lp1, lp2)

    out_ref[...] = jnp.concatenate([ent[:, None], log_prob[:, None]], axis=1)


@jax.jit
def kernel(logits_1, logits_2, value):
    B, V = logits_1.shape
    R = 16
    grid = (B // R,)
    v2d = value.astype(jnp.int32).reshape(B, 1)
    out = pl.pallas_call(
        _block_kernel,
        grid=grid,
        in_specs=[
            pl.BlockSpec((R, V), lambda i: (i, 0)),
            pl.BlockSpec((R, V), lambda i: (i, 0)),
            pl.BlockSpec((R, 1), lambda i: (i, 0)),
        ],
        out_specs=pl.BlockSpec((R, 2), lambda i: (i, 0)),
        out_shape=jax.ShapeDtypeStruct((B, 2), jnp.float32),
    )(logits_1, logits_2, v2d)
    return out
